# baseline (speedup 1.0000x reference)
import functools

import jax
import jax.numpy as jnp
from jax import lax
from jax.experimental import pallas as pl
from jax.experimental.pallas import tpu as pltpu
from jax.experimental.pallas import tpu_sc as plsc

_G = 2
_NBUF = 3


def kernel(arr):
    B, S2, D = arr.shape
    S = int(round(S2 ** 0.5))
    assert S * S == S2
    out_rows = S2 - S
    nblk = S - 1
    L = 16
    KPB = S // L

    nfull = nblk // _G
    rem = nblk - nfull * _G
    W = ((_G - 1) * (S + 1) + (nfull - 1) * _G + 1 + S + S - 1) // S * S
    t_d0 = nfull * _G
    t_src0 = (S + 1) * t_d0 + 1
    t_A = t_src0 - t_src0 % S
    t_offs = [(S + 1) * (t_d0 + j) + 1 - t_A for j in range(rem)]
    t_W = ((t_offs[-1] + S + S - 1) // S) * S
    assert t_A + t_W <= S2 and t_W <= W

    ntrip = nfull // _NBUF

    mesh = plsc.VectorSubcoreMesh(core_axis_name="c", subcore_axis_name="s")

    @functools.partial(
        pl.kernel,
        mesh=mesh,
        out_type=jax.ShapeDtypeStruct((B, D, out_rows), arr.dtype),
        scratch_types=(
            [pltpu.VMEM((D, W), arr.dtype) for _ in range(_NBUF)]
            + [pltpu.VMEM((D, S * _G), arr.dtype) for _ in range(_NBUF)]
            + [pltpu.SemaphoreType.DMA] * (2 * _NBUF)
        ),
        compiler_params=pltpu.CompilerParams(needs_layout_passes=False),
    )
    def copy_offdiag(arr_hbm, out_hbm, *refs):
        wins = refs[:_NBUF]
        outs = refs[_NBUF:2 * _NBUF]
        sem_g = refs[2 * _NBUF:3 * _NBUF]
        sem_s = refs[3 * _NBUF:]
        wid = lax.axis_index("s") * 2 + lax.axis_index("c")
        b = wid
        iota = lax.iota(jnp.int32, L)

        def gather(ci, p):
            pltpu.async_copy(
                arr_hbm.at[b, :, pl.ds(_G * S * ci, W)],
                wins[p],
                sem_g[p],
            )

        def wait_gather(ci, p):
            pltpu.make_async_copy(
                arr_hbm.at[b, :, pl.ds(_G * S * ci, W)],
                wins[p],
                sem_g[p],
            ).wait()

        def compact(ci, p):
            base = _G * ci + 1

            def row_body(r, carry):
                rvec = jnp.zeros((L,), jnp.int32) + r
                for j in range(_G):
                    for k in range(KPB):
                        src = iota + (base + (S + 1) * j + L * k)
                        v = plsc.load_gather(wins[p], [rvec, src])
                        outs[p][r, pl.ds(S * j + L * k, L)] = v
                return carry

            lax.fori_loop(0, D, row_body, None)

        def scatter(ci, p):
            pltpu.async_copy(
                outs[p],
                out_hbm.at[b, :, pl.ds(_G * S * ci, S * _G)],
                sem_s[p],
            )

        def wait_scatter(ci, p):
            pltpu.make_async_copy(
                outs[p],
                out_hbm.at[b, :, pl.ds(_G * S * ci, S * _G)],
                sem_s[p],
            ).wait()

        for p in range(_NBUF):
            gather(p, p)

        def trip_body(t, carry):
            for par in range(_NBUF):
                ci = _NBUF * t + par
                wait_gather(ci, par)
                compact(ci, par)
                scatter(ci, par)
                nc = ci + _NBUF

                @pl.when(nc <= nfull - 1)
                def _():
                    wait_scatter(ci, par)
                    gather(nc, par)

            return carry

        lax.fori_loop(0, ntrip, trip_body, None)

        for ci in range(ntrip * _NBUF, nfull):
            par = ci % _NBUF
            wait_gather(ci, par)
            compact(ci, par)
            scatter(ci, par)

        t_par = nfull % _NBUF
        t_prev = t_par + ((nfull - 1 - t_par) // _NBUF) * _NBUF
        wait_scatter(t_prev, t_par)
        pltpu.async_copy(
            arr_hbm.at[b, :, pl.ds(t_A, t_W)],
            wins[t_par].at[:, pl.ds(0, t_W)],
            sem_g[t_par],
        )
        pltpu.make_async_copy(
            arr_hbm.at[b, :, pl.ds(t_A, t_W)],
            wins[t_par].at[:, pl.ds(0, t_W)],
            sem_g[t_par],
        ).wait()

        def t_row_body(r, carry):
            rvec = jnp.zeros((L,), jnp.int32) + r
            for j in range(rem):
                for k in range(KPB):
                    src = iota + (t_offs[j] + L * k)
                    v = plsc.load_gather(wins[t_par], [rvec, src])
                    outs[t_par][r, pl.ds(S * j + L * k, L)] = v
            return carry

        lax.fori_loop(0, D, t_row_body, None)
        pltpu.async_copy(
            outs[t_par].at[:, pl.ds(0, S * rem)],
            out_hbm.at[b, :, pl.ds(S * t_d0, S * rem)],
            sem_s[t_par],
        )

        for ci in range(nfull - _NBUF, nfull):
            par = ci % _NBUF
            if par == t_par:
                continue
            wait_scatter(ci, par)
        pltpu.make_async_copy(
            outs[t_par].at[:, pl.ds(0, S * rem)],
            out_hbm.at[b, :, pl.ds(S * t_d0, S * rem)],
            sem_s[t_par],
        ).wait()

    arr_t = jnp.transpose(arr, (0, 2, 1))
    out_t = copy_offdiag(arr_t)
    return jnp.transpose(out_t, (0, 2, 1))


# speedup vs baseline: 2.4296x; 2.4296x over previous
import functools

import jax
import jax.numpy as jnp
from jax import lax
from jax.experimental import pallas as pl
from jax.experimental.pallas import tpu as pltpu
from jax.experimental.pallas import tpu_sc as plsc

_G = 2
_NBUF = 3


def kernel(arr):
    B, S2, D = arr.shape
    S = int(round(S2 ** 0.5))
    assert S * S == S2
    out_rows = S2 - S
    nblk = S - 1
    L = 16
    KPB = S // L
    OW = S * _G

    nfull = nblk // _G
    rem = nblk - nfull * _G
    W = ((_G - 1) * (S + 1) + (nfull - 1) * _G + 1 + S + S - 1) // S * S
    t_d0 = nfull * _G
    t_src0 = (S + 1) * t_d0 + 1
    t_A = t_src0 - t_src0 % S
    t_offs = [(S + 1) * (t_d0 + j) + 1 - t_A for j in range(rem)]
    t_W = ((t_offs[-1] + S + S - 1) // S) * S
    assert t_A + t_W <= S2 and t_W <= W

    ntrip = nfull // _NBUF

    mesh = plsc.VectorSubcoreMesh(core_axis_name="c", subcore_axis_name="s")

    @functools.partial(
        pl.kernel,
        mesh=mesh,
        out_type=jax.ShapeDtypeStruct((B, D, out_rows), arr.dtype),
        scratch_types=(
            [pltpu.VMEM((D, W), arr.dtype) for _ in range(_NBUF)]
            + [pltpu.VMEM((D, OW), arr.dtype) for _ in range(_NBUF)]
            + [pltpu.SemaphoreType.DMA] * (2 * _NBUF)
        ),
        compiler_params=pltpu.CompilerParams(
            needs_layout_passes=False, disable_bounds_checks=True),
    )
    def copy_offdiag(arr_hbm, out_hbm, *refs):
        wins = refs[:_NBUF]
        outs = refs[_NBUF:2 * _NBUF]
        sem_g = refs[2 * _NBUF:3 * _NBUF]
        sem_s = refs[3 * _NBUF:]
        wid = lax.axis_index("s") * 2 + lax.axis_index("c")
        b = wid
        iota = lax.iota(jnp.int32, L)

        def gather(ci, p):
            pltpu.async_copy(
                arr_hbm.at[b, :, pl.ds(_G * S * ci, W)],
                wins[p],
                sem_g[p],
            )

        def wait_gather(ci, p):
            pltpu.make_async_copy(
                arr_hbm.at[b, :, pl.ds(_G * S * ci, W)],
                wins[p],
                sem_g[p],
            ).wait()

        def compact_rows(win, out_ref, offs):
            @plsc.parallel_loop(0, D, step=1, unroll=2)
            def _(r):
                rvec = jnp.zeros((L,), jnp.int32) + r
                for j in range(len(offs)):
                    for k in range(KPB):
                        src = iota + (offs[j] + L * k)
                        v = plsc.load_gather(win, [rvec, src])
                        out_ref[r, pl.ds(S * j + L * k, L)] = v

        def compact(ci, p):
            base = _G * ci + 1
            offs = [base + (S + 1) * j for j in range(_G)]
            compact_rows(wins[p], outs[p], offs)

        def scatter(ci, p):
            pltpu.async_copy(
                outs[p],
                out_hbm.at[b, :, pl.ds(_G * S * ci, OW)],
                sem_s[p],
            )

        def wait_scatter(ci, p):
            pltpu.make_async_copy(
                outs[p],
                out_hbm.at[b, :, pl.ds(_G * S * ci, OW)],
                sem_s[p],
            ).wait()

        for p in range(_NBUF):
            gather(p, p)

        def trip_body(t, carry):
            for par in range(_NBUF):
                ci = _NBUF * t + par
                wait_gather(ci, par)
                compact(ci, par)
                scatter(ci, par)
                nc = ci + _NBUF

                @pl.when(nc <= nfull - 1)
                def _():
                    wait_scatter(ci, par)
                    gather(nc, par)

            return carry

        lax.fori_loop(0, ntrip, trip_body, None)

        for ci in range(ntrip * _NBUF, nfull):
            par = ci % _NBUF
            wait_gather(ci, par)
            compact(ci, par)
            scatter(ci, par)

        t_par = nfull % _NBUF
        t_prev = t_par + ((nfull - 1 - t_par) // _NBUF) * _NBUF
        wait_scatter(t_prev, t_par)
        pltpu.async_copy(
            arr_hbm.at[b, :, pl.ds(t_A, t_W)],
            wins[t_par].at[:, pl.ds(0, t_W)],
            sem_g[t_par],
        )
        pltpu.make_async_copy(
            arr_hbm.at[b, :, pl.ds(t_A, t_W)],
            wins[t_par].at[:, pl.ds(0, t_W)],
            sem_g[t_par],
        ).wait()

        compact_rows(wins[t_par], outs[t_par], t_offs)
        pltpu.async_copy(
            outs[t_par].at[:, pl.ds(0, S * rem)],
            out_hbm.at[b, :, pl.ds(S * t_d0, S * rem)],
            sem_s[t_par],
        )

        for ci in range(nfull - _NBUF, nfull):
            par = ci % _NBUF
            if par == t_par:
                continue
            wait_scatter(ci, par)
        pltpu.make_async_copy(
            outs[t_par].at[:, pl.ds(0, S * rem)],
            out_hbm.at[b, :, pl.ds(S * t_d0, S * rem)],
            sem_s[t_par],
        ).wait()

    arr_t = jnp.transpose(arr, (0, 2, 1))
    out_t = copy_offdiag(arr_t)
    return jnp.transpose(out_t, (0, 2, 1))
